# baseline (speedup 1.0000x reference)
import functools

import jax
import jax.numpy as jnp
import numpy as np
from jax import lax
from jax.experimental import pallas as pl
from jax.experimental.pallas import tpu as pltpu
from jax.experimental.pallas import tpu_sc as plsc

NUM_CORES = 2
NUM_SUBCORES = 16
NW = NUM_CORES * NUM_SUBCORES

B = 4096
L = 200
LP = 208
LPAD = 256
V1 = 1000001
D = 64
C = 10
BPW = B // NW

NBUF = 4

_S2T = np.concatenate([
    2 * np.arange(16), 2 * np.arange(16) + 1,
    32 + 2 * np.arange(16), 32 + 2 * np.arange(16) + 1,
])

_mesh = plsc.VectorSubcoreMesh(core_axis_name="c", subcore_axis_name="s")


@functools.partial(
    pl.kernel,
    out_type=jax.ShapeDtypeStruct((B * LPAD,), jnp.int32),
    mesh=_mesh,
    scratch_types=[pltpu.SemaphoreType.DMA],
    compiler_params=pltpu.CompilerParams(use_tc_tiling_on_sc=True),
)
def _sc_format_idx(idx_hbm, out_hbm, sem):
    wid = lax.axis_index("s") * NUM_CORES + lax.axis_index("c")
    base = wid * BPW

    def row_copy(b):
        return pltpu.make_async_copy(
            idx_hbm.at[base + b],
            out_hbm.at[pl.ds((base + b) * LPAD, LPAD)],
            sem,
        )

    def fire(b, carry):
        row_copy(b).start()
        return carry

    lax.fori_loop(0, BPW, fire, 0)

    def drain(b, carry):
        row_copy(b).wait()
        return carry

    lax.fori_loop(0, BPW, drain, 0)


@functools.partial(
    pl.kernel,
    out_type=jax.ShapeDtypeStruct((B, D), jnp.float32),
    mesh=_mesh,
    scratch_types=[
        pltpu.VMEM((BPW * LPAD,), jnp.int32),
        pltpu.VMEM((NBUF, LP, 32), jnp.int32),
        pltpu.VMEM((BPW, D), jnp.float32),
        pltpu.SemaphoreType.DMA,
        [pltpu.SemaphoreType.DMA] * NBUF,
    ],
    compiler_params=pltpu.CompilerParams(use_tc_tiling_on_sc=False),
)
def _sc_gather_sum(idx_hbm, table_hbm, out_hbm, idx_v, rows_v, sums_v,
                   sem_i, sems):
    wid = lax.axis_index("s") * NUM_CORES + lax.axis_index("c")
    base = wid * BPW

    pltpu.async_copy(idx_hbm.at[pl.ds(base * LPAD, BPW * LPAD)], idx_v,
                     sem_i).wait()

    def gather_row(b, buf):
        copies = []
        for k in range(LP // 16):
            idx16 = idx_v[pl.ds(b * LPAD + k * 16, 16)]
            copies.append(pltpu.make_async_copy(
                table_hbm.at[idx16],
                rows_v.at[buf, pl.ds(k * 16, 16)],
                sems[buf],
            ))
        return copies

    hi_mask = jnp.full((16,), -65536, jnp.int32)

    def accum(b, buf):
        def body(j4, acc):
            j = j4 * 4
            a0, a1, a2, a3 = acc
            for dj in range(4):
                w0 = rows_v[buf, j + dj, pl.ds(0, 16)]
                w1 = rows_v[buf, j + dj, pl.ds(16, 16)]
                a0 = a0 + lax.bitcast_convert_type(
                    lax.shift_left(w0, 16), jnp.float32)
                a1 = a1 + lax.bitcast_convert_type(w0 & hi_mask, jnp.float32)
                a2 = a2 + lax.bitcast_convert_type(
                    lax.shift_left(w1, 16), jnp.float32)
                a3 = a3 + lax.bitcast_convert_type(w1 & hi_mask, jnp.float32)
            return (a0, a1, a2, a3)

        zero = jnp.zeros((16,), jnp.float32)
        acc = lax.fori_loop(0, LP // 4, body, (zero, zero, zero, zero))
        for c in range(4):
            sums_v[b, pl.ds(c * 16, 16)] = acc[c]

    for p in range(NBUF):
        for d in gather_row(p, p):
            d.start()

    def step(k, carry):
        b0 = k * NBUF
        for p in range(NBUF):
            b = b0 + p
            for d in gather_row(b, p):
                d.wait()
            accum(b, p)

            @pl.when(b + NBUF < BPW)
            def _():
                for d in gather_row(b + NBUF, p):
                    d.start()

        return carry

    lax.fori_loop(0, BPW // NBUF, step, 0)
    pltpu.sync_copy(sums_v, out_hbm.at[pl.ds(base, BPW)])


def _tc_head_body(inp_ref, sums_ref, emb0_ref, w_ref, b_ref, out_ref):
    cnt = jnp.sum((inp_ref[...] != 0).astype(jnp.float32), axis=1,
                  keepdims=True)
    n_zero = jnp.float32(LP) - cnt
    pooled = (sums_ref[...] - n_zero * emb0_ref[...]) / jnp.maximum(cnt, 1.0)
    logits = jnp.dot(pooled, w_ref[...],
                     preferred_element_type=jnp.float32) + b_ref[...]
    m = jnp.max(logits, axis=-1, keepdims=True)
    e = jnp.exp(logits - m)
    out_ref[...] = e / jnp.sum(e, axis=-1, keepdims=True)


_tc_head = pl.pallas_call(
    _tc_head_body,
    out_shape=jax.ShapeDtypeStruct((B, C), jnp.float32),
)


def _pack_table(emb_table):
    bits = lax.bitcast_convert_type(emb_table, jnp.uint32)
    r = bits + 0x7FFF + ((bits >> 16) & 1)
    hi16 = r & jnp.uint32(0xFFFF0000)
    ev, od = hi16[:, 0::2], hi16[:, 1::2]
    return lax.bitcast_convert_type((ev >> 16) | od, jnp.int32)


def kernel(inputs, emb_table, W, b):
    idx_pad = jnp.pad(inputs, ((0, 0), (0, LPAD - L)))
    idx_flat = idx_pad.reshape(-1)
    table_q = _pack_table(emb_table)
    sums = _sc_gather_sum(idx_flat, table_q)
    emb0 = emb_table[0].astype(jnp.bfloat16).astype(jnp.float32)
    emb0_s = jnp.take(emb0, _S2T).reshape(1, D)
    w_s = W[_S2T, :]
    return _tc_head(inputs, sums, emb0_s, w_s,
                    b.reshape(1, C).astype(jnp.float32))


# speedup vs baseline: 5.7791x; 5.7791x over previous
import functools

import jax
import jax.numpy as jnp
import numpy as np
from jax import lax
from jax.experimental import pallas as pl
from jax.experimental.pallas import tpu as pltpu
from jax.experimental.pallas import tpu_sc as plsc

NUM_CORES = 2
NUM_SUBCORES = 16
NW = NUM_CORES * NUM_SUBCORES

B = 4096
L = 200
LP = 208
LPAD = 256
V1 = 1000001
D = 64
C = 10
BPW = B // NW

NBUF = 4

_S2T = np.concatenate([
    np.arange(16), 32 + np.arange(16),
    16 + np.arange(16), 48 + np.arange(16),
])

_mesh = plsc.VectorSubcoreMesh(core_axis_name="c", subcore_axis_name="s")


@functools.partial(
    pl.kernel,
    out_type=jax.ShapeDtypeStruct((B * LPAD,), jnp.int32),
    mesh=_mesh,
    scratch_types=[pltpu.SemaphoreType.DMA],
    compiler_params=pltpu.CompilerParams(use_tc_tiling_on_sc=True),
)
def _sc_format_idx(idx_hbm, out_hbm, sem):
    wid = lax.axis_index("s") * NUM_CORES + lax.axis_index("c")
    base = wid * BPW

    def row_copy(b):
        return pltpu.make_async_copy(
            idx_hbm.at[base + b],
            out_hbm.at[pl.ds((base + b) * LPAD, LPAD)],
            sem,
        )

    def fire(b, carry):
        row_copy(b).start()
        return carry

    lax.fori_loop(0, BPW, fire, 0)

    def drain(b, carry):
        row_copy(b).wait()
        return carry

    lax.fori_loop(0, BPW, drain, 0)


@functools.partial(
    pl.kernel,
    out_type=jax.ShapeDtypeStruct((B, D), jnp.float32),
    mesh=_mesh,
    scratch_types=[
        pltpu.VMEM((BPW * LPAD,), jnp.int32),
        pltpu.VMEM((NBUF, LP, 32), jnp.int32),
        pltpu.VMEM((BPW, D), jnp.float32),
        pltpu.SemaphoreType.DMA,
        [pltpu.SemaphoreType.DMA] * NBUF,
    ],
    compiler_params=pltpu.CompilerParams(use_tc_tiling_on_sc=False),
)
def _sc_gather_sum(idx_hbm, table_hbm, out_hbm, idx_v, rows_v, sums_v,
                   sem_i, sems):
    wid = lax.axis_index("s") * NUM_CORES + lax.axis_index("c")
    base = wid * BPW

    pltpu.async_copy(idx_hbm.at[pl.ds(base * LPAD, BPW * LPAD)], idx_v,
                     sem_i).wait()

    def gather_row(b, buf):
        copies = []
        for k in range(LP // 16):
            idx16 = idx_v[pl.ds(b * LPAD + k * 16, 16)]
            copies.append(pltpu.make_async_copy(
                table_hbm.at[idx16],
                rows_v.at[buf, pl.ds(k * 16, 16)],
                sems[buf],
            ))
        return copies

    hi_mask = jnp.full((16,), -65536, jnp.int32)

    def accum(b, buf):
        def body(j4, acc):
            j = j4 * 4
            a0, a1, a2, a3 = acc
            for dj in range(4):
                w0 = rows_v[buf, j + dj, pl.ds(0, 16)]
                w1 = rows_v[buf, j + dj, pl.ds(16, 16)]
                a0 = a0 + lax.bitcast_convert_type(
                    lax.shift_left(w0, 16), jnp.float32)
                a1 = a1 + lax.bitcast_convert_type(w0 & hi_mask, jnp.float32)
                a2 = a2 + lax.bitcast_convert_type(
                    lax.shift_left(w1, 16), jnp.float32)
                a3 = a3 + lax.bitcast_convert_type(w1 & hi_mask, jnp.float32)
            return (a0, a1, a2, a3)

        zero = jnp.zeros((16,), jnp.float32)
        acc = lax.fori_loop(0, LP // 4, body, (zero, zero, zero, zero))
        for c in range(4):
            sums_v[b, pl.ds(c * 16, 16)] = acc[c]

    for p in range(NBUF):
        for d in gather_row(p, p):
            d.start()

    def step(k, carry):
        b0 = k * NBUF
        for p in range(NBUF):
            b = b0 + p
            for d in gather_row(b, p):
                d.wait()
            accum(b, p)

            @pl.when(b + NBUF < BPW)
            def _():
                for d in gather_row(b + NBUF, p):
                    d.start()

        return carry

    lax.fori_loop(0, BPW // NBUF, step, 0)
    pltpu.sync_copy(sums_v, out_hbm.at[pl.ds(base, BPW)])


def _tc_head_body(inp_ref, sums_ref, emb0_ref, w_ref, b_ref, out_ref):
    cnt = jnp.sum((inp_ref[...] != 0).astype(jnp.float32), axis=1,
                  keepdims=True)
    n_zero = jnp.float32(LP) - cnt
    pooled = (sums_ref[...] - n_zero * emb0_ref[...]) / jnp.maximum(cnt, 1.0)
    logits = jnp.dot(pooled, w_ref[...],
                     preferred_element_type=jnp.float32) + b_ref[...]
    m = jnp.max(logits, axis=-1, keepdims=True)
    e = jnp.exp(logits - m)
    out_ref[...] = e / jnp.sum(e, axis=-1, keepdims=True)


_tc_head = pl.pallas_call(
    _tc_head_body,
    out_shape=jax.ShapeDtypeStruct((B, C), jnp.float32),
)


def _pack_table(emb_table):
    bits = lax.bitcast_convert_type(emb_table, jnp.uint32)
    r = bits + 0x7FFF + ((bits >> 16) & 1)
    hi16 = r & jnp.uint32(0xFFFF0000)
    lo, hi = hi16[:, :32], hi16[:, 32:]
    return lax.bitcast_convert_type((lo >> 16) | hi, jnp.int32)


def kernel(inputs, emb_table, W, b):
    idx_pad = jnp.pad(inputs, ((0, 0), (0, LPAD - L)))
    idx_flat = idx_pad.reshape(-1)
    table_q = _pack_table(emb_table)
    sums = _sc_gather_sum(idx_flat, table_q)
    emb0 = emb_table[0].astype(jnp.bfloat16).astype(jnp.float32)
    emb0_s = jnp.take(emb0, _S2T).reshape(1, D)
    w_s = W[_S2T, :]
    return _tc_head(inputs, sums, emb0_s, w_s,
                    b.reshape(1, C).astype(jnp.float32))


# speedup vs baseline: 7.9404x; 1.3740x over previous
import functools

import jax
import jax.numpy as jnp
from jax import lax
from jax.experimental import pallas as pl
from jax.experimental.pallas import tpu as pltpu
from jax.experimental.pallas import tpu_sc as plsc

NUM_CORES = 2
NUM_SUBCORES = 16
NW = NUM_CORES * NUM_SUBCORES

B = 4096
L = 200
LP = 208
LPAD = 256
D = 64
C = 10
BPW = B // NW

NBUF = 4

_mesh = plsc.VectorSubcoreMesh(core_axis_name="c", subcore_axis_name="s")


@functools.partial(
    pl.kernel,
    out_type=jax.ShapeDtypeStruct((B * LPAD,), jnp.int32),
    mesh=_mesh,
    scratch_types=[pltpu.SemaphoreType.DMA],
    compiler_params=pltpu.CompilerParams(use_tc_tiling_on_sc=True),
)
def _sc_format_idx(idx_hbm, out_hbm, sem):
    wid = lax.axis_index("s") * NUM_CORES + lax.axis_index("c")
    base = wid * BPW

    def row_copy(b):
        return pltpu.make_async_copy(
            idx_hbm.at[base + b],
            out_hbm.at[pl.ds((base + b) * LPAD, LPAD)],
            sem,
        )

    def fire(b, carry):
        row_copy(b).start()
        return carry

    lax.fori_loop(0, BPW, fire, 0)

    def drain(b, carry):
        row_copy(b).wait()
        return carry

    lax.fori_loop(0, BPW, drain, 0)


@functools.partial(
    pl.kernel,
    out_type=jax.ShapeDtypeStruct((B, D), jnp.float32),
    mesh=_mesh,
    scratch_types=[
        pltpu.VMEM((BPW * LPAD,), jnp.int32),
        pltpu.VMEM((NBUF, LP, D), jnp.float32),
        pltpu.VMEM((BPW, D), jnp.float32),
        pltpu.SemaphoreType.DMA,
        [pltpu.SemaphoreType.DMA] * NBUF,
    ],
    compiler_params=pltpu.CompilerParams(use_tc_tiling_on_sc=False),
)
def _sc_gather_sum(idx_hbm, table_hbm, out_hbm, idx_v, rows_v, sums_v,
                   sem_i, sems):
    wid = lax.axis_index("s") * NUM_CORES + lax.axis_index("c")
    base = wid * BPW

    pltpu.async_copy(idx_hbm.at[pl.ds(base * LPAD, BPW * LPAD)], idx_v,
                     sem_i).wait()

    def gather_row(b, buf):
        copies = []
        for k in range(LP // 16):
            idx16 = idx_v[pl.ds(b * LPAD + k * 16, 16)]
            copies.append(pltpu.make_async_copy(
                table_hbm.at[idx16],
                rows_v.at[buf, pl.ds(k * 16, 16)],
                sems[buf],
            ))
        return copies

    def accum(b, buf):
        def body(j4, acc):
            j = j4 * 4
            for dj in range(4):
                acc = tuple(
                    acc[c] + rows_v[buf, j + dj, pl.ds(c * 16, 16)]
                    for c in range(4)
                )
            return acc

        zero = jnp.zeros((16,), jnp.float32)
        acc = lax.fori_loop(0, LP // 4, body, (zero, zero, zero, zero))
        for c in range(4):
            sums_v[b, pl.ds(c * 16, 16)] = acc[c]

    for p in range(NBUF):
        for d in gather_row(p, p):
            d.start()

    def step(k, carry):
        b0 = k * NBUF
        for p in range(NBUF):
            b = b0 + p
            for d in gather_row(b, p):
                d.wait()
            accum(b, p)

            @pl.when(b + NBUF < BPW)
            def _():
                for d in gather_row(b + NBUF, p):
                    d.start()

        return carry

    lax.fori_loop(0, BPW // NBUF, step, 0)
    pltpu.sync_copy(sums_v, out_hbm.at[pl.ds(base, BPW)])


def _tc_head_body(inp_ref, sums_ref, emb0_ref, w_ref, b_ref, out_ref):
    cnt = jnp.sum((inp_ref[...] != 0).astype(jnp.float32), axis=1,
                  keepdims=True)
    n_zero = jnp.float32(LP) - cnt
    pooled = (sums_ref[...] - n_zero * emb0_ref[...]) / jnp.maximum(cnt, 1.0)
    logits = jnp.dot(pooled, w_ref[...],
                     preferred_element_type=jnp.float32) + b_ref[...]
    m = jnp.max(logits, axis=-1, keepdims=True)
    e = jnp.exp(logits - m)
    out_ref[...] = e / jnp.sum(e, axis=-1, keepdims=True)


_tc_head = pl.pallas_call(
    _tc_head_body,
    out_shape=jax.ShapeDtypeStruct((B, C), jnp.float32),
)


def kernel(inputs, emb_table, W, b):
    idx_pad = jnp.pad(inputs, ((0, 0), (0, LPAD - L)))
    idx_flat = _sc_format_idx(idx_pad)
    sums = _sc_gather_sum(idx_flat, emb_table)
    return _tc_head(inputs, sums, emb_table[0:1], W,
                    b.reshape(1, C).astype(jnp.float32))


# speedup vs baseline: 7.9804x; 1.0050x over previous
import functools

import jax
import jax.numpy as jnp
from jax import lax
from jax.experimental import pallas as pl
from jax.experimental.pallas import tpu as pltpu
from jax.experimental.pallas import tpu_sc as plsc

NUM_CORES = 2
NUM_SUBCORES = 16
NW = NUM_CORES * NUM_SUBCORES

B = 4096
L = 200
LP = 208
LPAD = 256
D = 64
C = 10
BPW = B // NW

NBUF = 4

_mesh = plsc.VectorSubcoreMesh(core_axis_name="c", subcore_axis_name="s")


@functools.partial(
    pl.kernel,
    out_type=jax.ShapeDtypeStruct((B, D), jnp.float32),
    mesh=_mesh,
    scratch_types=[
        pltpu.VMEM((BPW, LPAD), jnp.int32),
        pltpu.VMEM((NBUF, LP, D), jnp.float32),
        pltpu.VMEM((BPW, D), jnp.float32),
        pltpu.SemaphoreType.DMA,
        [pltpu.SemaphoreType.DMA] * NBUF,
    ],
    compiler_params=pltpu.CompilerParams(use_tc_tiling_on_sc=False),
)
def _sc_gather_sum(idx_hbm, table_hbm, out_hbm, idx_v, rows_v, sums_v,
                   sem_i, sems):
    wid = lax.axis_index("s") * NUM_CORES + lax.axis_index("c")
    base = wid * BPW

    pltpu.async_copy(idx_hbm.at[pl.ds(base, BPW)], idx_v, sem_i).wait()

    def gather_row(b, buf):
        return [
            pltpu.make_async_copy(
                table_hbm.at[idx_v[b, pl.ds(k * 16, 16)]],
                rows_v.at[buf, pl.ds(k * 16, 16)],
                sems[buf],
            )
            for k in range(LP // 16)
        ]

    def accum(b, buf):
        def body(j4, acc):
            j = j4 * 4
            for dj in range(4):
                acc = tuple(
                    acc[c] + rows_v[buf, j + dj, pl.ds(c * 16, 16)]
                    for c in range(4)
                )
            return acc

        zero = jnp.zeros((16,), jnp.float32)
        acc = lax.fori_loop(0, LP // 4, body, (zero, zero, zero, zero))
        for c in range(4):
            sums_v[b, pl.ds(c * 16, 16)] = acc[c]

    for p in range(NBUF):
        for d in gather_row(p, p):
            d.start()

    def step(k, carry):
        b0 = k * NBUF
        for p in range(NBUF):
            b = b0 + p
            for d in gather_row(b, p):
                d.wait()
            accum(b, p)

            @pl.when(b + NBUF < BPW)
            def _():
                for d in gather_row(b + NBUF, p):
                    d.start()

        return carry

    lax.fori_loop(0, BPW // NBUF, step, 0)
    pltpu.sync_copy(sums_v, out_hbm.at[pl.ds(base, BPW)])


def _tc_head_body(inp_ref, sums_ref, emb0_ref, w_ref, b_ref, out_ref):
    cnt = jnp.sum((inp_ref[...] != 0).astype(jnp.float32), axis=1,
                  keepdims=True)
    n_zero = jnp.float32(LP) - cnt
    pooled = (sums_ref[...] - n_zero * emb0_ref[...]) / jnp.maximum(cnt, 1.0)
    logits = jnp.dot(pooled, w_ref[...],
                     preferred_element_type=jnp.float32) + b_ref[...]
    m = jnp.max(logits, axis=-1, keepdims=True)
    e = jnp.exp(logits - m)
    out_ref[...] = e / jnp.sum(e, axis=-1, keepdims=True)


_tc_head = pl.pallas_call(
    _tc_head_body,
    out_shape=jax.ShapeDtypeStruct((B, C), jnp.float32),
)


def kernel(inputs, emb_table, W, b):
    idx_pad = jnp.pad(inputs, ((0, 0), (0, LPAD - L)))
    sums = _sc_gather_sum(idx_pad, emb_table)
    return _tc_head(inputs, sums, emb_table[0:1], W,
                    b.reshape(1, C).astype(jnp.float32))
